# baseline (speedup 1.0000x reference)
import functools

import jax
import jax.numpy as jnp
from jax import lax
from jax.experimental import pallas as pl
from jax.experimental.pallas import tpu as pltpu
from jax.experimental.pallas import tpu_sc as plsc

_L = 16


def _entry_body(uv_ref, x_ref, up_ref, out0_ref, out1_ref):
    B = uv_ref.shape[0]
    P = uv_ref.shape[1]
    C = x_ref.shape[1]
    HW = x_ref.shape[2]
    q = lax.broadcasted_iota(jnp.int32, (P, HW), 1)
    outs = (out0_ref, out1_ref)
    for b in range(B):
        uvb = uv_ref[b]
        g = jnp.clip((uvb - 0.5) * 2.0, -1.0, 1.0)
        gx = (g[:, 0:1] + 1.0) * 31.5
        gy = (g[:, 1:2] + 1.0) * 31.5
        x0 = jnp.floor(gx)
        y0 = jnp.floor(gy)
        wx1 = gx - x0
        wy1 = gy - y0
        x0i = jnp.clip(x0, 0.0, 63.0).astype(jnp.int32)
        x1i = jnp.clip(x0 + 1.0, 0.0, 63.0).astype(jnp.int32)
        y0i = jnp.clip(y0, 0.0, 63.0).astype(jnp.int32)
        y1i = jnp.clip(y0 + 1.0, 0.0, 63.0).astype(jnp.int32)

        def tap(yi, xi, w):
            return jnp.where(q == yi * 64 + xi, w, 0.0)

        S = (tap(y0i, x0i, (1.0 - wy1) * (1.0 - wx1))
             + tap(y0i, x1i, (1.0 - wy1) * wx1)
             + tap(y1i, x0i, wy1 * (1.0 - wx1))
             + tap(y1i, x1i, wy1 * wx1))
        featT = lax.dot_general(S, x_ref[b], (((1,), (1,)), ((), ())),
                                preferred_element_type=jnp.float32)
        m = jnp.dot(up_ref[...], featT, preferred_element_type=jnp.float32)
        outs[b // 2][:, pl.ds((b % 2) * C, C)] = m


def _entry(uv, xflat, up):
    B, P, _ = uv.shape
    C = xflat.shape[1]
    HW = xflat.shape[2]
    V4 = up.shape[0]
    return pl.pallas_call(
        _entry_body,
        in_specs=[
            pl.BlockSpec((B, P, 2), lambda: (0, 0, 0)),
            pl.BlockSpec((B, C, HW), lambda: (0, 0, 0)),
            pl.BlockSpec((V4, P), lambda: (0, 0)),
        ],
        out_specs=[pl.BlockSpec((V4, 2 * C), lambda: (0, 0))] * 2,
        out_shape=[jax.ShapeDtypeStruct((V4, 2 * C), jnp.float32)] * 2,
    )(uv, xflat, up)


def _mm_body(a_ref, b_ref, o_ref, *, relu):
    r = jnp.dot(a_ref[...], b_ref[...], preferred_element_type=jnp.float32)
    o_ref[...] = jnp.maximum(r, 0.0) if relu else r


def _mm(a, bT, relu, bm=2048):
    M, K = a.shape
    N = bT.shape[1]
    return pl.pallas_call(
        functools.partial(_mm_body, relu=relu),
        grid=(M // bm,),
        in_specs=[
            pl.BlockSpec((bm, K), lambda i: (i, 0)),
            pl.BlockSpec((K, N), lambda i: (0, 0)),
        ],
        out_specs=pl.BlockSpec((bm, N), lambda i: (i, 0)),
        out_shape=jax.ShapeDtypeStruct((M, N), jnp.float32),
    )(a, bT)


_VSUB = {3: 16, 9: 8}
_CHUNK = {
    (9, 512): 8, (9, 256): 16, (9, 128): 32, (9, 64): 64,
    (3, 512): 16, (3, 256): 48, (3, 128): 96,
}


def _sc_stage(tables, idxflat, w, K, D, Vout, mode):
    chains = len(tables)
    info = plsc.get_sparse_core_info()
    NC, NS = info.num_cores, info.num_subcores
    NW = NC * NS
    per_w = Vout // NW
    n = _CHUNK[(K, D)]
    vsub = _VSUB[K]
    nsub = n // vsub
    nchunks = per_w // n
    assert per_w % n == 0 and n % vsub == 0
    if chains == 1:
        assert nchunks % 2 == 0
    nlanes = D // _L

    if mode == "spiral":
        w_scratch = pltpu.VMEM((K, D), jnp.float32)
    else:
        w_scratch = pltpu.VMEM((per_w * K + _L,), jnp.float32)

    @functools.partial(
        pl.kernel,
        out_type=[jax.ShapeDtypeStruct((Vout, D), jnp.float32)] * chains,
        mesh=plsc.VectorSubcoreMesh(core_axis_name="c", subcore_axis_name="s"),
        scratch_types=[
            pltpu.VMEM((per_w * K,), jnp.int32),
            pltpu.VMEM((n * K, D), jnp.float32),
            pltpu.VMEM((n * K, D), jnp.float32),
            pltpu.VMEM((n, D), jnp.float32),
            pltpu.VMEM((n, D), jnp.float32),
            w_scratch,
            pltpu.SemaphoreType.DMA,
            pltpu.SemaphoreType.DMA,
            pltpu.SemaphoreType.DMA,
            pltpu.SemaphoreType.DMA,
        ],
    )
    def k(*refs):
        t_hbm = refs[:chains]
        idx_hbm = refs[chains]
        w_hbm = refs[chains + 1]
        out_hbm = refs[chains + 2:2 * chains + 2]
        (idx_all, buf0, buf1, acc0, acc1, wv,
         semg0, semg1, sems0, sems1) = refs[2 * chains + 2:]
        bufs = (buf0, buf1)
        accs = (acc0, acc1)
        semg = (semg0, semg1)
        sems = (sems0, sems1)
        wid = lax.axis_index("s") * NC + lax.axis_index("c")
        base0 = wid * per_w
        pltpu.sync_copy(idx_hbm.at[pl.ds(base0 * K, per_w * K)], idx_all)
        if mode == "spiral":
            pltpu.sync_copy(w_hbm, wv)
        else:
            pltpu.sync_copy(w_hbm.at[pl.ds(base0 * K, per_w * K)],
                            wv.at[pl.ds(0, per_w * K)])

        def fire(b, ch, ci):
            for s in range(nsub):
                off = ci * n * K + s * vsub * K
                pltpu.async_copy(
                    t_hbm[ch].at[idx_all.at[pl.ds(off, vsub * K)]],
                    bufs[b].at[pl.ds(s * vsub * K, vsub * K)],
                    semg[b])

        def drain_gather(b):
            pltpu.make_async_copy(out_hbm[0].at[pl.ds(0, n * K)], bufs[b],
                                  semg[b]).wait()

        def drain_store(b):
            pltpu.make_async_copy(accs[b], out_hbm[0].at[pl.ds(base0, n)],
                                  sems[b]).wait()

        def compute(b, ci):
            buf = bufs[b]
            acc = accs[b]
            if mode == "spiral":
                def lane_body(j, c2):
                    sl = pl.ds(j * _L, _L)
                    wregs = [wv[s, sl] for s in range(K)]
                    for i in range(n):
                        a = wregs[0] * buf[i * K, sl]
                        for s in range(1, K):
                            a = a + wregs[s] * buf[i * K + s, sl]
                        acc[i, sl] = a
                    return c2
                lax.fori_loop(0, nlanes, lane_body, 0)
            else:
                def row_body(i, c2):
                    wvec = wv[pl.ds((ci * n + i) * K, _L)]
                    wregs = [wvec[s] for s in range(K)]
                    for j in range(nlanes):
                        sl = pl.ds(j * _L, _L)
                        a = wregs[0] * buf[i * K, sl]
                        for s in range(1, K):
                            a = a + wregs[s] * buf[i * K + s, sl]
                        acc[i, sl] = a
                    return c2
                lax.fori_loop(0, n, row_body, 0)

        fire(0, 0, 0)

        def outer(cc, carry):
            for b in range(2):
                ch = b % chains
                cur = cc if chains == 2 else cc * 2 + b
                if chains == 2 and b == 0:
                    fire(1, 1, cc)
                else:
                    nxt = cc + 1 if chains == 2 else cc * 2 + b + 1

                    @pl.when(nxt < nchunks)
                    def _():
                        fire(1 - b, 0 if chains == 1 else 0, nxt)

                drain_gather(b)

                @pl.when(cc >= 1)
                def _():
                    drain_store(b)

                compute(b, cur)
                pltpu.async_copy(accs[b],
                                 out_hbm[ch].at[pl.ds(base0 + cur * n, n)],
                                 sems[b])
            return carry

        lax.fori_loop(0, nchunks if chains == 2 else nchunks // 2, outer, 0)
        drain_store(0)
        drain_store(1)

    return k(*tables, idxflat, w)


def kernel(uv, x, upsample, dw0, pw0, dw1, pw1, dw2, pw2, dw3, pw3, dwh, pwh,
           sp0, sp1, sp2, sp3,
           row0, col0, val0, row1, col1, val1, row2, col2, val2,
           row3, col3, val3):
    B, C0 = x.shape[0], x.shape[1]
    xflat = x.reshape(B, C0, x.shape[2] * x.shape[3])
    hs = list(_entry(uv, xflat, upsample))

    levels = [
        (col3, val3, sp3, dw0, pw0),
        (col2, val2, sp2, dw1, pw1),
        (col1, val1, sp1, dw2, pw2),
        (col0, val0, sp0, dw3, pw3),
    ]
    for col, val, sp, dwl, pwl in levels:
        Cin = dwl.shape[0]
        D = 2 * Cin
        Vout = col.shape[0] // 3
        Cout = pwl.shape[0]
        dwt = jnp.tile(dwl.T[:, None, :], (1, 2, 1)).reshape(sp.shape[1], D)
        pooled = _sc_stage(tuple(hs), col, val, 3, D, Vout, "pool")
        gg = _sc_stage(tuple(pooled), sp.reshape(-1), dwt, sp.shape[1], D,
                       Vout, "spiral")
        for g in range(2):
            o = _mm(gg[g].reshape(Vout * 2, Cin), pwl.T, relu=True)
            hs[g] = o.reshape(Vout, 2 * Cout)

    Cin = dwh.shape[0]
    D = B * Cin
    V0 = sp0.shape[0]
    merged = jnp.concatenate(hs, axis=1)
    dwt = jnp.tile(dwh.T[:, None, :], (1, B, 1)).reshape(sp0.shape[1], D)
    gh = _sc_stage((merged,), sp0.reshape(-1), dwt, sp0.shape[1], D, V0,
                   "spiral")[0]
    pred = _mm(gh.reshape(V0 * B, Cin), pwh.T, relu=False)
    return pred.reshape(V0, B, 3).transpose(1, 0, 2)


# speedup vs baseline: 1.1032x; 1.1032x over previous
import functools

import jax
import jax.numpy as jnp
from jax import lax
from jax.experimental import pallas as pl
from jax.experimental.pallas import tpu as pltpu
from jax.experimental.pallas import tpu_sc as plsc

_L = 16


def _entry_body(uv_ref, x_ref, up_ref, out0_ref, out1_ref):
    B = uv_ref.shape[0]
    P = uv_ref.shape[1]
    C = x_ref.shape[1]
    HW = x_ref.shape[2]
    q = lax.broadcasted_iota(jnp.int32, (P, HW), 1)
    outs = (out0_ref, out1_ref)
    for b in range(B):
        uvb = uv_ref[b]
        g = jnp.clip((uvb - 0.5) * 2.0, -1.0, 1.0)
        gx = (g[:, 0:1] + 1.0) * 31.5
        gy = (g[:, 1:2] + 1.0) * 31.5
        x0 = jnp.floor(gx)
        y0 = jnp.floor(gy)
        wx1 = gx - x0
        wy1 = gy - y0
        x0i = jnp.clip(x0, 0.0, 63.0).astype(jnp.int32)
        x1i = jnp.clip(x0 + 1.0, 0.0, 63.0).astype(jnp.int32)
        y0i = jnp.clip(y0, 0.0, 63.0).astype(jnp.int32)
        y1i = jnp.clip(y0 + 1.0, 0.0, 63.0).astype(jnp.int32)

        def tap(yi, xi, w):
            return jnp.where(q == yi * 64 + xi, w, 0.0)

        S = (tap(y0i, x0i, (1.0 - wy1) * (1.0 - wx1))
             + tap(y0i, x1i, (1.0 - wy1) * wx1)
             + tap(y1i, x0i, wy1 * (1.0 - wx1))
             + tap(y1i, x1i, wy1 * wx1))
        featT = lax.dot_general(S, x_ref[b], (((1,), (1,)), ((), ())),
                                preferred_element_type=jnp.float32,
                                precision=lax.Precision.HIGHEST)
        m = jnp.dot(up_ref[...], featT, preferred_element_type=jnp.float32,
                    precision=lax.Precision.HIGHEST)
        outs[b // 2][:, pl.ds((b % 2) * C, C)] = m


def _entry(uv, xflat, up):
    B, P, _ = uv.shape
    C = xflat.shape[1]
    HW = xflat.shape[2]
    V4 = up.shape[0]
    return pl.pallas_call(
        _entry_body,
        in_specs=[
            pl.BlockSpec((B, P, 2), lambda: (0, 0, 0)),
            pl.BlockSpec((B, C, HW), lambda: (0, 0, 0)),
            pl.BlockSpec((V4, P), lambda: (0, 0)),
        ],
        out_specs=[pl.BlockSpec((V4, 2 * C), lambda: (0, 0))] * 2,
        out_shape=[jax.ShapeDtypeStruct((V4, 2 * C), jnp.float32)] * 2,
    )(uv, xflat, up)


def _mm_body(a_ref, b_ref, o_ref, *, relu):
    r = jnp.dot(a_ref[...], b_ref[...], preferred_element_type=jnp.float32)
    o_ref[...] = jnp.maximum(r, 0.0) if relu else r


def _mm(a, bT, relu, bm=2048):
    M, K = a.shape
    N = bT.shape[1]
    return pl.pallas_call(
        functools.partial(_mm_body, relu=relu),
        grid=(M // bm,),
        in_specs=[
            pl.BlockSpec((bm, K), lambda i: (i, 0)),
            pl.BlockSpec((K, N), lambda i: (0, 0)),
        ],
        out_specs=pl.BlockSpec((bm, N), lambda i: (i, 0)),
        out_shape=jax.ShapeDtypeStruct((M, N), jnp.float32),
    )(a, bT)


_VSUB = {3: 16, 9: 8}
_CHUNK = {
    (9, 512): 8, (9, 256): 16, (9, 128): 32, (9, 64): 64,
    (3, 512): 16, (3, 256): 48, (3, 128): 96,
}


def _sc_stage(tables, idxflat, w, K, D, Vout, mode):
    chains = len(tables)
    info = plsc.get_sparse_core_info()
    NC, NS = info.num_cores, info.num_subcores
    NW = NC * NS
    per_w = Vout // NW
    n = _CHUNK[(K, D)]
    vsub = _VSUB[K]
    nsub = n // vsub
    nchunks = per_w // n
    assert per_w % n == 0 and n % vsub == 0
    if chains == 1:
        assert nchunks % 2 == 0
    nlanes = D // _L

    if mode == "spiral":
        w_scratch = pltpu.VMEM((K, D), jnp.float32)
    else:
        w_scratch = pltpu.VMEM((per_w * K + _L,), jnp.float32)

    @functools.partial(
        pl.kernel,
        out_type=[jax.ShapeDtypeStruct((Vout, D), jnp.float32)] * chains,
        mesh=plsc.VectorSubcoreMesh(core_axis_name="c", subcore_axis_name="s"),
        scratch_types=[
            pltpu.VMEM((per_w * K,), jnp.int32),
            pltpu.VMEM((n * K, D), jnp.float32),
            pltpu.VMEM((n * K, D), jnp.float32),
            pltpu.VMEM((n, D), jnp.float32),
            pltpu.VMEM((n, D), jnp.float32),
            w_scratch,
            pltpu.SemaphoreType.DMA,
            pltpu.SemaphoreType.DMA,
            pltpu.SemaphoreType.DMA,
            pltpu.SemaphoreType.DMA,
        ],
    )
    def k(*refs):
        t_hbm = refs[:chains]
        idx_hbm = refs[chains]
        w_hbm = refs[chains + 1]
        out_hbm = refs[chains + 2:2 * chains + 2]
        (idx_all, buf0, buf1, acc0, acc1, wv,
         semg0, semg1, sems0, sems1) = refs[2 * chains + 2:]
        bufs = (buf0, buf1)
        accs = (acc0, acc1)
        semg = (semg0, semg1)
        sems = (sems0, sems1)
        wid = lax.axis_index("s") * NC + lax.axis_index("c")
        base0 = wid * per_w
        pltpu.sync_copy(idx_hbm.at[pl.ds(base0 * K, per_w * K)], idx_all)
        if mode == "spiral":
            pltpu.sync_copy(w_hbm, wv)
        else:
            pltpu.sync_copy(w_hbm.at[pl.ds(base0 * K, per_w * K)],
                            wv.at[pl.ds(0, per_w * K)])

        def fire(b, ch, ci):
            for s in range(nsub):
                off = ci * n * K + s * vsub * K
                pltpu.async_copy(
                    t_hbm[ch].at[idx_all.at[pl.ds(off, vsub * K)]],
                    bufs[b].at[pl.ds(s * vsub * K, vsub * K)],
                    semg[b])

        def drain_gather(b):
            pltpu.make_async_copy(out_hbm[0].at[pl.ds(0, n * K)], bufs[b],
                                  semg[b]).wait()

        def drain_store(b):
            pltpu.make_async_copy(accs[b], out_hbm[0].at[pl.ds(base0, n)],
                                  sems[b]).wait()

        def compute(b, ci):
            buf = bufs[b]
            acc = accs[b]
            if mode == "spiral":
                def lane_body(j, c2):
                    sl = pl.ds(j * _L, _L)
                    wregs = [wv[s, sl] for s in range(K)]
                    for i in range(n):
                        a = wregs[0] * buf[i * K, sl]
                        for s in range(1, K):
                            a = a + wregs[s] * buf[i * K + s, sl]
                        acc[i, sl] = a
                    return c2
                lax.fori_loop(0, nlanes, lane_body, 0)
            else:
                def row_body(i, c2):
                    wvec = wv[pl.ds((ci * n + i) * K, _L)]
                    wregs = [wvec[s] for s in range(K)]
                    for j in range(nlanes):
                        sl = pl.ds(j * _L, _L)
                        a = wregs[0] * buf[i * K, sl]
                        for s in range(1, K):
                            a = a + wregs[s] * buf[i * K + s, sl]
                        acc[i, sl] = a
                    return c2
                lax.fori_loop(0, n, row_body, 0)

        fire(0, 0, 0)

        def outer(cc, carry):
            for b in range(2):
                ch = b % chains
                cur = cc if chains == 2 else cc * 2 + b
                if chains == 2 and b == 0:
                    fire(1, 1, cc)
                else:
                    nxt = cc + 1 if chains == 2 else cc * 2 + b + 1

                    @pl.when(nxt < nchunks)
                    def _():
                        fire(1 - b, 0 if chains == 1 else 0, nxt)

                drain_gather(b)

                @pl.when(cc >= 1)
                def _():
                    drain_store(b)

                compute(b, cur)
                pltpu.async_copy(accs[b],
                                 out_hbm[ch].at[pl.ds(base0 + cur * n, n)],
                                 sems[b])
            return carry

        lax.fori_loop(0, nchunks if chains == 2 else nchunks // 2, outer, 0)
        drain_store(0)
        drain_store(1)

    return k(*tables, idxflat, w)


def kernel(uv, x, upsample, dw0, pw0, dw1, pw1, dw2, pw2, dw3, pw3, dwh, pwh,
           sp0, sp1, sp2, sp3,
           row0, col0, val0, row1, col1, val1, row2, col2, val2,
           row3, col3, val3):
    B, C0 = x.shape[0], x.shape[1]
    xflat = x.reshape(B, C0, x.shape[2] * x.shape[3])
    hs = list(_entry(uv, xflat, upsample))

    levels = [
        (col3, val3, sp3, dw0, pw0),
        (col2, val2, sp2, dw1, pw1),
        (col1, val1, sp1, dw2, pw2),
        (col0, val0, sp0, dw3, pw3),
    ]
    for col, val, sp, dwl, pwl in levels:
        Cin = dwl.shape[0]
        D = 2 * Cin
        Vout = col.shape[0] // 3
        Cout = pwl.shape[0]
        dwt = jnp.tile(dwl.T[:, None, :], (1, 2, 1)).reshape(sp.shape[1], D)
        for g in range(2):
            pooled = _sc_stage((hs[g],), col, val, 3, D, Vout, "pool")[0]
            gg = _sc_stage((pooled,), sp.reshape(-1), dwt, sp.shape[1], D,
                           Vout, "spiral")[0]
            o = _mm(gg.reshape(Vout * 2, Cin), pwl.T, relu=True)
            hs[g] = o.reshape(Vout, 2 * Cout)

    Cin = dwh.shape[0]
    D = B * Cin
    V0 = sp0.shape[0]
    merged = jnp.concatenate(hs, axis=1)
    dwt = jnp.tile(dwh.T[:, None, :], (1, B, 1)).reshape(sp0.shape[1], D)
    gh = _sc_stage((merged,), sp0.reshape(-1), dwt, sp0.shape[1], D, V0,
                   "spiral")[0]
    pred = _mm(gh.reshape(V0 * B, Cin), pwh.T, relu=False)
    return pred.reshape(V0, B, 3).transpose(1, 0, 2)


# speedup vs baseline: 1.1762x; 1.0662x over previous
import functools

import jax
import jax.numpy as jnp
from jax import lax
from jax.experimental import pallas as pl
from jax.experimental.pallas import tpu as pltpu
from jax.experimental.pallas import tpu_sc as plsc

_L = 16


def _entry_body(uv_ref, x_ref, up_ref, out0_ref, out1_ref):
    B = uv_ref.shape[0]
    P = uv_ref.shape[1]
    C = x_ref.shape[1]
    HW = x_ref.shape[2]
    q = lax.broadcasted_iota(jnp.int32, (P, HW), 1)
    outs = (out0_ref, out1_ref)
    for b in range(B):
        uvb = uv_ref[b]
        g = jnp.clip((uvb - 0.5) * 2.0, -1.0, 1.0)
        gx = (g[:, 0:1] + 1.0) * 31.5
        gy = (g[:, 1:2] + 1.0) * 31.5
        x0 = jnp.floor(gx)
        y0 = jnp.floor(gy)
        wx1 = gx - x0
        wy1 = gy - y0
        x0i = jnp.clip(x0, 0.0, 63.0).astype(jnp.int32)
        x1i = jnp.clip(x0 + 1.0, 0.0, 63.0).astype(jnp.int32)
        y0i = jnp.clip(y0, 0.0, 63.0).astype(jnp.int32)
        y1i = jnp.clip(y0 + 1.0, 0.0, 63.0).astype(jnp.int32)

        def tap(yi, xi, w):
            return jnp.where(q == yi * 64 + xi, w, 0.0)

        S = (tap(y0i, x0i, (1.0 - wy1) * (1.0 - wx1))
             + tap(y0i, x1i, (1.0 - wy1) * wx1)
             + tap(y1i, x0i, wy1 * (1.0 - wx1))
             + tap(y1i, x1i, wy1 * wx1))
        featT = lax.dot_general(S, x_ref[b], (((1,), (1,)), ((), ())),
                                preferred_element_type=jnp.float32,
                                precision=lax.Precision.HIGHEST)
        m = jnp.dot(up_ref[...], featT, preferred_element_type=jnp.float32,
                    precision=lax.Precision.HIGHEST)
        outs[b // 2][:, pl.ds((b % 2) * C, C)] = m


def _entry(uv, xflat, up):
    B, P, _ = uv.shape
    C = xflat.shape[1]
    HW = xflat.shape[2]
    V4 = up.shape[0]
    return pl.pallas_call(
        _entry_body,
        in_specs=[
            pl.BlockSpec((B, P, 2), lambda: (0, 0, 0)),
            pl.BlockSpec((B, C, HW), lambda: (0, 0, 0)),
            pl.BlockSpec((V4, P), lambda: (0, 0)),
        ],
        out_specs=[pl.BlockSpec((V4, 2 * C), lambda: (0, 0))] * 2,
        out_shape=[jax.ShapeDtypeStruct((V4, 2 * C), jnp.float32)] * 2,
    )(uv, xflat, up)


def _mm_body(a_ref, b_ref, o_ref, *, relu):
    r = jnp.dot(a_ref[...], b_ref[...], preferred_element_type=jnp.float32)
    r = jnp.maximum(r, 0.0) if relu else r
    o_ref[...] = r.astype(o_ref.dtype)


def _mm(a, bT, relu, out_dtype=jnp.float32, bm=2048):
    M, K = a.shape
    N = bT.shape[1]
    return pl.pallas_call(
        functools.partial(_mm_body, relu=relu),
        grid=(M // bm,),
        in_specs=[
            pl.BlockSpec((bm, K), lambda i: (i, 0)),
            pl.BlockSpec((K, N), lambda i: (0, 0)),
        ],
        out_specs=pl.BlockSpec((bm, N), lambda i: (i, 0)),
        out_shape=jax.ShapeDtypeStruct((M, N), out_dtype),
    )(a, bT)


_VSUB = {3: 16, 9: 8}
_CHUNK = {
    (9, 512): 8, (9, 256): 16, (9, 128): 32, (9, 64): 64,
    (3, 512): 16, (3, 256): 48, (3, 128): 96,
}


_HI = jnp.uint32(0xFFFF0000)
_RN = jnp.uint32(0x7FFF)


def _pack2(a, b):
    au = lax.bitcast_convert_type(a, jnp.uint32)
    bu = lax.bitcast_convert_type(b, jnp.uint32)
    ra = au + (((au >> 16) & jnp.uint32(1)) + _RN)
    rb = bu + (((bu >> 16) & jnp.uint32(1)) + _RN)
    return lax.bitcast_convert_type((ra >> 16) | (rb & _HI), jnp.int32)


def _unpack2(word):
    wu = lax.bitcast_convert_type(word, jnp.uint32)
    a = lax.bitcast_convert_type(wu << 16, jnp.float32)
    b = lax.bitcast_convert_type(wu & _HI, jnp.float32)
    return a, b


def _sc_stage(tables, idxflat, w, K, D, Vout, mode, packed=False):
    chains = len(tables)
    info = plsc.get_sparse_core_info()
    NC, NS = info.num_cores, info.num_subcores
    NW = NC * NS
    per_w = Vout // NW
    n = _CHUNK[(K, D)]
    if packed and mode == "spiral":
        n *= 2
    vsub = _VSUB[K]
    nsub = n // vsub
    nchunks = per_w // n
    assert per_w % n == 0 and n % vsub == 0
    if chains == 1:
        assert nchunks % 2 == 0
    nlanes = D // _L

    if mode == "spiral":
        w_scratch = pltpu.VMEM((K, D), jnp.float32)
        in_w = D // 2 if packed else D
        out_w, out_dtype = D, jnp.float32
        in_dtype = jnp.int32 if packed else jnp.float32
    else:
        w_scratch = pltpu.VMEM((per_w * K + _L,), jnp.float32)
        in_w, in_dtype = D, jnp.float32
        out_w = D // 2 if packed else D
        out_dtype = jnp.int32 if packed else jnp.float32

    @functools.partial(
        pl.kernel,
        out_type=[jax.ShapeDtypeStruct((Vout, out_w), out_dtype)] * chains,
        mesh=plsc.VectorSubcoreMesh(core_axis_name="c", subcore_axis_name="s"),
        scratch_types=[
            pltpu.VMEM((per_w * K,), jnp.int32),
            pltpu.VMEM((n * K, in_w), in_dtype),
            pltpu.VMEM((n * K, in_w), in_dtype),
            pltpu.VMEM((n, out_w), out_dtype),
            pltpu.VMEM((n, out_w), out_dtype),
            w_scratch,
            pltpu.SemaphoreType.DMA,
            pltpu.SemaphoreType.DMA,
            pltpu.SemaphoreType.DMA,
            pltpu.SemaphoreType.DMA,
        ],
    )
    def k(*refs):
        t_hbm = refs[:chains]
        idx_hbm = refs[chains]
        w_hbm = refs[chains + 1]
        out_hbm = refs[chains + 2:2 * chains + 2]
        (idx_all, buf0, buf1, acc0, acc1, wv,
         semg0, semg1, sems0, sems1) = refs[2 * chains + 2:]
        bufs = (buf0, buf1)
        accs = (acc0, acc1)
        semg = (semg0, semg1)
        sems = (sems0, sems1)
        wid = lax.axis_index("s") * NC + lax.axis_index("c")
        base0 = wid * per_w
        pltpu.sync_copy(idx_hbm.at[pl.ds(base0 * K, per_w * K)], idx_all)
        if mode == "spiral":
            pltpu.sync_copy(w_hbm, wv)
        else:
            pltpu.sync_copy(w_hbm.at[pl.ds(base0 * K, per_w * K)],
                            wv.at[pl.ds(0, per_w * K)])

        def fire(b, ch, ci):
            for s in range(nsub):
                off = ci * n * K + s * vsub * K
                pltpu.async_copy(
                    t_hbm[ch].at[idx_all.at[pl.ds(off, vsub * K)]],
                    bufs[b].at[pl.ds(s * vsub * K, vsub * K)],
                    semg[b])

        def drain_gather(b):
            pltpu.make_async_copy(t_hbm[0].at[pl.ds(0, n * K)], bufs[b],
                                  semg[b]).wait()

        def drain_store(b):
            pltpu.make_async_copy(accs[b], out_hbm[0].at[pl.ds(base0, n)],
                                  sems[b]).wait()

        def compute(b, ci):
            buf = bufs[b]
            acc = accs[b]
            if mode == "spiral" and packed:
                def lane_body(m, c2):
                    sla = pl.ds(m * 2 * _L, _L)
                    slb = pl.ds(m * 2 * _L + _L, _L)
                    wa = [wv[s, sla] for s in range(K)]
                    wb = [wv[s, slb] for s in range(K)]
                    for i in range(n):
                        a0, b0 = _unpack2(buf[i * K, pl.ds(m * _L, _L)])
                        a = wa[0] * a0
                        bb = wb[0] * b0
                        for s in range(1, K):
                            a1, b1 = _unpack2(
                                buf[i * K + s, pl.ds(m * _L, _L)])
                            a = a + wa[s] * a1
                            bb = bb + wb[s] * b1
                        acc[i, sla] = a
                        acc[i, slb] = bb
                    return c2
                lax.fori_loop(0, nlanes // 2, lane_body, 0)
            elif mode == "spiral":
                def lane_body(j, c2):
                    sl = pl.ds(j * _L, _L)
                    wregs = [wv[s, sl] for s in range(K)]
                    for i in range(n):
                        a = wregs[0] * buf[i * K, sl]
                        for s in range(1, K):
                            a = a + wregs[s] * buf[i * K + s, sl]
                        acc[i, sl] = a
                    return c2
                lax.fori_loop(0, nlanes, lane_body, 0)
            else:
                def row_body(i, c2):
                    wvec = wv[pl.ds((ci * n + i) * K, _L)]
                    wregs = [wvec[s] for s in range(K)]

                    def tap(off):
                        a = wregs[0] * buf[i * K, off]
                        for s in range(1, K):
                            a = a + wregs[s] * buf[i * K + s, off]
                        return a

                    if packed:
                        for m in range(nlanes // 2):
                            p = _pack2(tap(pl.ds(m * 2 * _L, _L)),
                                       tap(pl.ds(m * 2 * _L + _L, _L)))
                            acc[i, pl.ds(m * _L, _L)] = p
                    else:
                        for j in range(nlanes):
                            sl = pl.ds(j * _L, _L)
                            acc[i, sl] = tap(sl)
                    return c2
                lax.fori_loop(0, n, row_body, 0)

        fire(0, 0, 0)

        def outer(cc, carry):
            for b in range(2):
                ch = b % chains
                cur = cc if chains == 2 else cc * 2 + b
                if chains == 2 and b == 0:
                    fire(1, 1, cc)
                else:
                    nxt = cc + 1 if chains == 2 else cc * 2 + b + 1

                    @pl.when(nxt < nchunks)
                    def _():
                        fire(1 - b, 0 if chains == 1 else 0, nxt)

                drain_gather(b)

                @pl.when(cc >= 1)
                def _():
                    drain_store(b)

                compute(b, cur)
                pltpu.async_copy(accs[b],
                                 out_hbm[ch].at[pl.ds(base0 + cur * n, n)],
                                 sems[b])
            return carry

        lax.fori_loop(0, nchunks if chains == 2 else nchunks // 2, outer, 0)
        drain_store(0)
        drain_store(1)

    return k(*tables, idxflat, w)


def kernel(uv, x, upsample, dw0, pw0, dw1, pw1, dw2, pw2, dw3, pw3, dwh, pwh,
           sp0, sp1, sp2, sp3,
           row0, col0, val0, row1, col1, val1, row2, col2, val2,
           row3, col3, val3):
    B, C0 = x.shape[0], x.shape[1]
    xflat = x.reshape(B, C0, x.shape[2] * x.shape[3])
    hs = list(_entry(uv, xflat, upsample))

    levels = [
        (col3, val3, sp3, dw0, pw0),
        (col2, val2, sp2, dw1, pw1),
        (col1, val1, sp1, dw2, pw2),
        (col0, val0, sp0, dw3, pw3),
    ]
    for li, (col, val, sp, dwl, pwl) in enumerate(levels):
        Cin = dwl.shape[0]
        D = 2 * Cin
        Vout = col.shape[0] // 3
        Cout = pwl.shape[0]
        packed = 2 * Cin >= 256
        dwt = jnp.tile(dwl.T[:, None, :], (1, 2, 1)).reshape(sp.shape[1], D)
        for g in range(2):
            pooled = _sc_stage((hs[g],), col, val, 3, D, Vout, "pool",
                               packed=packed)[0]
            gg = _sc_stage((pooled,), sp.reshape(-1), dwt, sp.shape[1], D,
                           Vout, "spiral", packed=packed)[0]
            o = _mm(gg.reshape(Vout * 2, Cin), pwl.T, relu=True)
            hs[g] = o.reshape(Vout, 2 * Cout)

    Cin = dwh.shape[0]
    D = B * Cin
    V0 = sp0.shape[0]
    merged = jnp.concatenate(hs, axis=1)
    dwt = jnp.tile(dwh.T[:, None, :], (1, B, 1)).reshape(sp0.shape[1], D)
    gh = _sc_stage((merged,), sp0.reshape(-1), dwt, sp0.shape[1], D, V0,
                   "spiral")[0]
    pred = _mm(gh.reshape(V0 * B, Cin), pwh.T, relu=False)
    return pred.reshape(V0, B, 3).transpose(1, 0, 2)
